# baseline (speedup 1.0000x reference)
import functools

import jax
import jax.numpy as jnp
from jax import lax
from jax.experimental import pallas as pl
from jax.experimental.pallas import tpu as pltpu
from jax.experimental.pallas import tpu_sc as plsc

N = 10000
E = 320000
D = 128
NSC = 2
NH = N // NSC
NTILE = 16
LANES = 16
EB = 128
NB = 158
DEPTH = 2
E_PAD = NTILE * NB * EB
HT_ROWS = N + 8
PAD_SRC = N
PAD_DST = N + 1
TRASH = NH

CHUNK = 312
TAIL0 = CHUNK * NTILE
TAIL = NH - TAIL0

EB_DEG = 128
NB_DEG = 80
N_DEG = 10240
CHUNK_DEG = N_DEG // NTILE
PAD_DST_DEG = 10100

_mesh = plsc.VectorSubcoreMesh(core_axis_name="c", subcore_axis_name="s")


def _sc_deg_body(dstp_hbm, ones_hbm, zeros_hbm, out_hbm, dst_v, ones_v, acc):
    c = lax.axis_index("c")
    s = lax.axis_index("s")
    w = c * NTILE + s
    r0 = s * CHUNK_DEG
    pltpu.sync_copy(dstp_hbm.at[w], dst_v)
    pltpu.sync_copy(ones_hbm, ones_v)
    pltpu.sync_copy(zeros_hbm.at[pl.ds(r0, CHUNK_DEG)],
                    acc.at[pl.ds(r0, CHUNK_DEG)])
    plsc.subcore_barrier()

    @pl.loop(0, NB_DEG)
    def _(j):
        pltpu.sync_copy(ones_v, acc.at[dst_v.at[j]], add=True)

    plsc.subcore_barrier()
    pltpu.sync_copy(acc.at[pl.ds(r0, CHUNK_DEG)],
                    out_hbm.at[c, 0, pl.ds(r0, CHUNK_DEG)])


def _sc_deg(dstp, ones, zeros):
    return pl.kernel(
        _sc_deg_body,
        out_type=jax.ShapeDtypeStruct((NSC, 1, N_DEG), jnp.float32),
        mesh=_mesh,
        scratch_types=[
            pltpu.VMEM((NB_DEG, EB_DEG), jnp.int32),
            pltpu.VMEM((EB_DEG,), jnp.float32),
            pltpu.VMEM_SHARED((N_DEG + 128,), jnp.float32),
        ],
    )(dstp, ones, zeros)


def _sc_agg_body(ht_hbm, srcp_hbm, dstp_hbm, out_hbm, src_v, dst_v,
                 bufs, sems, acc):
    c = lax.axis_index("c")
    s = lax.axis_index("s")
    r0 = s * CHUNK
    nbase = c * NH
    pltpu.sync_copy(srcp_hbm.at[s], src_v)
    pltpu.sync_copy(dstp_hbm.at[s], dst_v)
    pltpu.sync_copy(
        ht_hbm.at[pl.ds(pl.multiple_of(nbase + r0, 8), CHUNK)],
        acc.at[pl.ds(r0, CHUNK)])

    @pl.when(s == NTILE - 1)
    def _():
        pltpu.sync_copy(
            ht_hbm.at[pl.ds(pl.multiple_of(nbase + TAIL0, 8), TAIL)],
            acc.at[pl.ds(TAIL0, TAIL)])

    base16 = jnp.full((LANES,), nbase, jnp.int32)
    trash16 = jnp.full((LANES,), TRASH, jnp.int32)
    zero16 = jnp.zeros((LANES,), jnp.int32)
    nh16 = jnp.full((LANES,), NH, jnp.int32)

    @pl.loop(0, NB)
    def _(r):
        @pl.loop(0, EB // LANES)
        def _(k):
            sl = pl.ds(k * LANES, LANES)
            t = dst_v[r, sl] - base16
            ok = (t >= zero16) & (t < nh16)
            dst_v[r, sl] = jnp.where(ok, t, trash16)

    plsc.subcore_barrier()

    def gather(j, b):
        return pltpu.make_async_copy(
            ht_hbm.at[src_v.at[j]], bufs.at[b], sems.at[b])

    for b in range(DEPTH):
        gather(b, b).start()

    @pl.loop(0, NB // DEPTH)
    def _(g):
        base = g * DEPTH
        for b in range(DEPTH):
            gather(base + b, b).wait()
            pltpu.sync_copy(bufs.at[b], acc.at[dst_v.at[base + b]], add=True)
            gather(base + DEPTH + b, b).start()

    for b in range(DEPTH):
        gather(NB + b, b).wait()

    plsc.subcore_barrier()
    pltpu.sync_copy(acc.at[pl.ds(r0, CHUNK)],
                    out_hbm.at[c, pl.ds(r0, CHUNK), :])

    @pl.when(s == NTILE - 1)
    def _():
        pltpu.sync_copy(acc.at[pl.ds(TAIL0, TAIL)],
                        out_hbm.at[c, pl.ds(TAIL0, TAIL), :])


def _sc_agg(ht, srcp, dstp):
    out = pl.kernel(
        _sc_agg_body,
        out_type=jax.ShapeDtypeStruct((NSC, NH, D), jnp.float32),
        mesh=_mesh,
        scratch_types=[
            pltpu.VMEM((NB + DEPTH, EB), jnp.int32),
            pltpu.VMEM((NB, EB), jnp.int32),
            pltpu.VMEM((DEPTH, EB, D), jnp.float32),
            pltpu.SemaphoreType.DMA((DEPTH,)),
            pltpu.VMEM_SHARED((NH + 8, D), jnp.float32),
        ],
    )(ht, srcp, dstp)
    return out.reshape(N, D)


_DOT = dict(precision=lax.Precision.HIGHEST, preferred_element_type=jnp.float32)


def _store_ht(ht_ref, h):
    ht_ref[pl.ds(0, N), :] = h
    ht_ref[pl.ds(N, HT_ROWS - N), :] = jnp.zeros((HT_ROWS - N, D),
                                                 jnp.float32)


def _tc_first_body(x_ref, w_ref, deg_ref, ht_ref, dinv_ref):
    deg = deg_ref[0] + deg_ref[1] + 1.0
    dinv = lax.rsqrt(deg)
    dinv_ref[...] = dinv
    _store_ht(ht_ref, jnp.dot(x_ref[...], w_ref[...], **_DOT) * dinv)


def _tc_first(x, W1, deg2):
    return pl.pallas_call(
        _tc_first_body,
        out_shape=(jax.ShapeDtypeStruct((HT_ROWS, D), jnp.float32),
                   jax.ShapeDtypeStruct((N, 1), jnp.float32)),
    )(x, W1, deg2)


def _tc_mid_body(agg_ref, dinv_ref, b_ref, w_ref, ht_ref):
    dinv = dinv_ref[...]
    t = jnp.maximum(agg_ref[...] * dinv + b_ref[...], 0.0)
    _store_ht(ht_ref, jnp.dot(t, w_ref[...], **_DOT) * dinv)


def _tc_mid(agg, dinv, b, W):
    return pl.pallas_call(
        _tc_mid_body,
        out_shape=jax.ShapeDtypeStruct((HT_ROWS, D), jnp.float32),
    )(agg, dinv, b, W)


def _tc_head_body(agg_ref, dinv_ref, b3_ref, wc1_ref, bc1_ref,
                  wc2_ref, bc2_ref, out_ref):
    h3 = jnp.maximum(agg_ref[...] * dinv_ref[...] + b3_ref[...], 0.0)
    pooled = jnp.sum(h3, axis=0, keepdims=True) * (1.0 / N)
    hidden = jnp.maximum(jnp.dot(pooled, wc1_ref[...], **_DOT) + bc1_ref[...],
                         0.0)
    out_ref[...] = jnp.dot(hidden, wc2_ref[...], **_DOT) + bc2_ref[...]


def _tc_head(agg, dinv, b3, Wc1, bc1, Wc2, bc2):
    return pl.pallas_call(
        _tc_head_body,
        out_shape=jax.ShapeDtypeStruct((1, 2), jnp.float32),
    )(agg, dinv, b3, Wc1, bc1, Wc2, bc2)


@jax.jit
def kernel(x, edge_index, W1, b1, W2, b2, W3, b3, Wc1, bc1, Wc2, bc2):
    src = edge_index[0].astype(jnp.int32)
    dst = edge_index[1].astype(jnp.int32)

    npad = E_PAD - E
    srcp = jnp.concatenate([src, jnp.full((npad,), PAD_SRC, jnp.int32)])
    dstp = jnp.concatenate([dst, jnp.full((npad,), PAD_DST, jnp.int32)])
    srcp = srcp.reshape(NTILE, NB, EB)
    dstp = dstp.reshape(NTILE, NB, EB)
    srcp = jnp.concatenate(
        [srcp, jnp.full((NTILE, DEPTH, EB), PAD_SRC, jnp.int32)], axis=1)

    npad_deg = NSC * NTILE * NB_DEG * EB_DEG - E
    dstp_deg = jnp.concatenate(
        [dst, jnp.full((npad_deg,), PAD_DST_DEG, jnp.int32)])
    dstp_deg = dstp_deg.reshape(NSC * NTILE, NB_DEG, EB_DEG)
    ones = jnp.ones((EB_DEG,), jnp.float32)
    zeros = jnp.zeros((N_DEG,), jnp.float32)

    deg2 = _sc_deg(dstp_deg, ones, zeros)[:, 0, :N].reshape(NSC, N, 1)
    ht1, dinv = _tc_first(x, W1, deg2)
    agg1 = _sc_agg(ht1, srcp, dstp)
    ht2 = _tc_mid(agg1, dinv, b1.reshape(1, D), W2)
    agg2 = _sc_agg(ht2, srcp, dstp)
    ht3 = _tc_mid(agg2, dinv, b2.reshape(1, D), W3)
    agg3 = _sc_agg(ht3, srcp, dstp)
    return _tc_head(agg3, dinv, b3.reshape(1, D), Wc1,
                    bc1.reshape(1, D), Wc2, bc2.reshape(1, 2))


# speedup vs baseline: 1.0653x; 1.0653x over previous
import functools

import jax
import jax.numpy as jnp
from jax import lax
from jax.experimental import pallas as pl
from jax.experimental.pallas import tpu as pltpu
from jax.experimental.pallas import tpu_sc as plsc

N = 10000
E = 320000
D = 128
NSC = 2
NH = N // NSC
NTILE = 16
LANES = 16
EB = 128
NB = 158
DEPTH = 2
E_PAD = NTILE * NB * EB
HT_ROWS = N + 8
PAD_SRC = N
PAD_DST = N + 1
NTRASH = 256

CHUNK = 312
TAIL0 = CHUNK * NTILE
TAIL = NH - TAIL0

EB_DEG = 128
NB_DEG = 80
N_DEG = 10240
CHUNK_DEG = N_DEG // NTILE
PAD_DST_DEG = 10100

_mesh = plsc.VectorSubcoreMesh(core_axis_name="c", subcore_axis_name="s")


def _sc_deg_body(dstp_hbm, ones_hbm, zeros_hbm, out_hbm, dst_v, ones_v, acc):
    c = lax.axis_index("c")
    s = lax.axis_index("s")
    w = c * NTILE + s
    r0 = s * CHUNK_DEG
    pltpu.sync_copy(dstp_hbm.at[w], dst_v)
    pltpu.sync_copy(ones_hbm, ones_v)
    pltpu.sync_copy(zeros_hbm.at[pl.ds(r0, CHUNK_DEG)],
                    acc.at[pl.ds(r0, CHUNK_DEG)])
    plsc.subcore_barrier()

    @pl.loop(0, NB_DEG)
    def _(j):
        pltpu.sync_copy(ones_v, acc.at[dst_v.at[j]], add=True)

    plsc.subcore_barrier()
    pltpu.sync_copy(acc.at[pl.ds(r0, CHUNK_DEG)],
                    out_hbm.at[c, 0, pl.ds(r0, CHUNK_DEG)])


def _sc_deg(dstp, ones, zeros):
    return pl.kernel(
        _sc_deg_body,
        out_type=jax.ShapeDtypeStruct((NSC, 1, N_DEG), jnp.float32),
        mesh=_mesh,
        scratch_types=[
            pltpu.VMEM((NB_DEG, EB_DEG), jnp.int32),
            pltpu.VMEM((EB_DEG,), jnp.float32),
            pltpu.VMEM_SHARED((N_DEG + 128,), jnp.float32),
        ],
    )(dstp, ones, zeros)


def _sc_agg_body(ht_hbm, srcp_hbm, dstp_hbm, out_hbm, src_v, dst_v,
                 bufs, sems, acc):
    c = lax.axis_index("c")
    s = lax.axis_index("s")
    r0 = s * CHUNK
    nbase = c * NH
    pltpu.sync_copy(srcp_hbm.at[s], src_v)
    pltpu.sync_copy(dstp_hbm.at[s], dst_v)
    pltpu.sync_copy(
        ht_hbm.at[pl.ds(pl.multiple_of(nbase + r0, 8), CHUNK)],
        acc.at[pl.ds(r0, CHUNK)])

    @pl.when(s == NTILE - 1)
    def _():
        pltpu.sync_copy(
            ht_hbm.at[pl.ds(pl.multiple_of(nbase + TAIL0, 8), TAIL)],
            acc.at[pl.ds(TAIL0, TAIL)])

    base16 = jnp.full((LANES,), nbase, jnp.int32)
    zero16 = jnp.zeros((LANES,), jnp.int32)
    nh16 = jnp.full((LANES,), NH, jnp.int32)
    tmask16 = jnp.full((LANES,), NTRASH - 1, jnp.int32)

    @pl.loop(0, NB)
    def _(r):
        @pl.loop(0, EB // LANES)
        def _(k):
            sl = pl.ds(k * LANES, LANES)
            d = dst_v[r, sl]
            t = d - base16
            ok = (t >= zero16) & (t < nh16)
            trash = nh16 + (d & tmask16)
            dst_v[r, sl] = jnp.where(ok, t, trash)

    plsc.subcore_barrier()

    def gather(j, b):
        return pltpu.make_async_copy(
            ht_hbm.at[src_v.at[j]], bufs.at[b], sems.at[b])

    for b in range(DEPTH):
        gather(b, b).start()

    @pl.loop(0, NB // DEPTH)
    def _(g):
        base = g * DEPTH
        for b in range(DEPTH):
            gather(base + b, b).wait()
            pltpu.sync_copy(bufs.at[b], acc.at[dst_v.at[base + b]], add=True)
            gather(base + DEPTH + b, b).start()

    for b in range(DEPTH):
        gather(NB + b, b).wait()

    plsc.subcore_barrier()
    pltpu.sync_copy(acc.at[pl.ds(r0, CHUNK)],
                    out_hbm.at[c, pl.ds(r0, CHUNK), :])

    @pl.when(s == NTILE - 1)
    def _():
        pltpu.sync_copy(acc.at[pl.ds(TAIL0, TAIL)],
                        out_hbm.at[c, pl.ds(TAIL0, TAIL), :])


def _sc_agg(ht, srcp, dstp):
    out = pl.kernel(
        _sc_agg_body,
        out_type=jax.ShapeDtypeStruct((NSC, NH, D), jnp.float32),
        mesh=_mesh,
        scratch_types=[
            pltpu.VMEM((NB + DEPTH, EB), jnp.int32),
            pltpu.VMEM((NB, EB), jnp.int32),
            pltpu.VMEM((DEPTH, EB, D), jnp.float32),
            pltpu.SemaphoreType.DMA((DEPTH,)),
            pltpu.VMEM_SHARED((NH + NTRASH, D), jnp.float32),
        ],
    )(ht, srcp, dstp)
    return out.reshape(N, D)


_DOT = dict(precision=lax.Precision.HIGHEST, preferred_element_type=jnp.float32)


def _store_ht(ht_ref, h):
    ht_ref[pl.ds(0, N), :] = h
    ht_ref[pl.ds(N, HT_ROWS - N), :] = jnp.zeros((HT_ROWS - N, D),
                                                 jnp.float32)


def _tc_first_body(x_ref, w_ref, deg_ref, ht_ref, dinv_ref):
    deg = deg_ref[0] + deg_ref[1] + 1.0
    dinv = lax.rsqrt(deg)
    dinv_ref[...] = dinv
    _store_ht(ht_ref, jnp.dot(x_ref[...], w_ref[...], **_DOT) * dinv)


def _tc_first(x, W1, deg2):
    return pl.pallas_call(
        _tc_first_body,
        out_shape=(jax.ShapeDtypeStruct((HT_ROWS, D), jnp.float32),
                   jax.ShapeDtypeStruct((N, 1), jnp.float32)),
    )(x, W1, deg2)


def _tc_mid_body(agg_ref, dinv_ref, b_ref, w_ref, ht_ref):
    dinv = dinv_ref[...]
    t = jnp.maximum(agg_ref[...] * dinv + b_ref[...], 0.0)
    _store_ht(ht_ref, jnp.dot(t, w_ref[...], **_DOT) * dinv)


def _tc_mid(agg, dinv, b, W):
    return pl.pallas_call(
        _tc_mid_body,
        out_shape=jax.ShapeDtypeStruct((HT_ROWS, D), jnp.float32),
    )(agg, dinv, b, W)


def _tc_head_body(agg_ref, dinv_ref, b3_ref, wc1_ref, bc1_ref,
                  wc2_ref, bc2_ref, out_ref):
    h3 = jnp.maximum(agg_ref[...] * dinv_ref[...] + b3_ref[...], 0.0)
    pooled = jnp.sum(h3, axis=0, keepdims=True) * (1.0 / N)
    hidden = jnp.maximum(jnp.dot(pooled, wc1_ref[...], **_DOT) + bc1_ref[...],
                         0.0)
    out_ref[...] = jnp.dot(hidden, wc2_ref[...], **_DOT) + bc2_ref[...]


def _tc_head(agg, dinv, b3, Wc1, bc1, Wc2, bc2):
    return pl.pallas_call(
        _tc_head_body,
        out_shape=jax.ShapeDtypeStruct((1, 2), jnp.float32),
    )(agg, dinv, b3, Wc1, bc1, Wc2, bc2)


@jax.jit
def kernel(x, edge_index, W1, b1, W2, b2, W3, b3, Wc1, bc1, Wc2, bc2):
    src = edge_index[0].astype(jnp.int32)
    dst = edge_index[1].astype(jnp.int32)

    npad = E_PAD - E
    srcp = jnp.concatenate([src, jnp.full((npad,), PAD_SRC, jnp.int32)])
    dstp = jnp.concatenate([dst, jnp.full((npad,), PAD_DST, jnp.int32)])
    srcp = srcp.reshape(NTILE, NB, EB)
    dstp = dstp.reshape(NTILE, NB, EB)
    srcp = jnp.concatenate(
        [srcp, jnp.full((NTILE, DEPTH, EB), PAD_SRC, jnp.int32)], axis=1)

    npad_deg = NSC * NTILE * NB_DEG * EB_DEG - E
    dstp_deg = jnp.concatenate(
        [dst, jnp.full((npad_deg,), PAD_DST_DEG, jnp.int32)])
    dstp_deg = dstp_deg.reshape(NSC * NTILE, NB_DEG, EB_DEG)
    ones = jnp.ones((EB_DEG,), jnp.float32)
    zeros = jnp.zeros((N_DEG,), jnp.float32)

    deg2 = _sc_deg(dstp_deg, ones, zeros)[:, 0, :N].reshape(NSC, N, 1)
    ht1, dinv = _tc_first(x, W1, deg2)
    agg1 = _sc_agg(ht1, srcp, dstp)
    ht2 = _tc_mid(agg1, dinv, b1.reshape(1, D), W2)
    agg2 = _sc_agg(ht2, srcp, dstp)
    ht3 = _tc_mid(agg2, dinv, b2.reshape(1, D), W3)
    agg3 = _sc_agg(ht3, srcp, dstp)
    return _tc_head(agg3, dinv, b3.reshape(1, D), Wc1,
                    bc1.reshape(1, D), Wc2, bc2.reshape(1, 2))
